# Initial kernel scaffold; baseline (speedup 1.0000x reference)
import jax
import jax.numpy as jnp
from jax.experimental import pallas as pl


def kernel(x, w1_oi, w2_oi, w1x1_oi, w1, w2, w1x1, b1, b2, b1x1, g1, be1, g2, be2):
    raise NotImplementedError("write your pallas kernel here")


# speedup vs baseline: 1.3966x; 1.3966x over previous
import jax
import jax.numpy as jnp
from jax import lax
from jax.experimental import pallas as pl
from jax.experimental.pallas import tpu as pltpu

_NB = 4


def _band_mats(wk, H, W):
    hi = jnp.arange(H)[:, None]
    ho = jnp.arange(H)[None, :]
    wi = jnp.arange(W)[:, None]
    wo = jnp.arange(W)[None, :]
    dh = hi - ho + 1
    dw = wi - wo + 1
    ok = ((dh >= 0) & (dh <= 2)).astype(wk.dtype)[:, :, None, None] * \
         ((dw >= 0) & (dw <= 2)).astype(wk.dtype)[None, None, :, :]
    m = wk[:, jnp.clip(dh, 0, 2)][:, :, :, jnp.clip(dw, 0, 2)]
    m = m * ok[None, :, :, :, :, None, None]
    m = jnp.transpose(m, (0, 1, 3, 5, 2, 4, 6))
    Ci, Co = wk.shape[3], wk.shape[4]
    return m.reshape(3, H * W * Ci, H * W * Co)


def _tap_cat(a):
    NB, D, L = a.shape
    z = jnp.zeros((NB, 1, L), a.dtype)
    up = jnp.concatenate([z, a[:, :-1]], axis=1)
    dn = jnp.concatenate([a[:, 1:], z], axis=1)
    return jnp.concatenate([up, a, dn], axis=-1).reshape(NB * D, 3 * L)


def _conv_stats_body(xb_ref, w3_ref, b_ref, y_ref, s_ref, q_ref):
    a3 = _tap_cat(xb_ref[...])
    acc = jnp.dot(a3, w3_ref[...], preferred_element_type=jnp.float32)
    acc = acc + b_ref[...]
    s_ref[0] = jnp.sum(acc, axis=0, keepdims=True)
    q_ref[0] = jnp.sum(acc * acc, axis=0, keepdims=True)
    NB, D, L = y_ref.shape
    y_ref[...] = acc.reshape(NB, D, L).astype(y_ref.dtype)


def _bnrelu_conv_stats_body(y1_ref, w3_ref, b_ref, sc_ref, sh_ref,
                            y_ref, s_ref, q_ref):
    a = jnp.maximum(y1_ref[...].astype(jnp.float32) * sc_ref[...] + sh_ref[...],
                    0.0).astype(jnp.bfloat16)
    a3 = _tap_cat(a)
    acc = jnp.dot(a3, w3_ref[...], preferred_element_type=jnp.float32)
    acc = acc + b_ref[...]
    s_ref[0] = jnp.sum(acc, axis=0, keepdims=True)
    q_ref[0] = jnp.sum(acc * acc, axis=0, keepdims=True)
    NB, D, L = y_ref.shape
    y_ref[...] = acc.reshape(NB, D, L).astype(y_ref.dtype)


def _epilogue_body(y2_ref, xb_ref, wr_ref, br_ref, sc_ref, sh_ref, o_ref):
    NB, D, L = o_ref.shape
    a2 = jnp.maximum(y2_ref[...].astype(jnp.float32) * sc_ref[...] + sh_ref[...],
                     0.0)
    res = jnp.dot(xb_ref[...].reshape(NB * D, L), wr_ref[...],
                  preferred_element_type=jnp.float32) + br_ref[...]
    o_ref[...] = jnp.maximum(a2 + res.reshape(NB, D, L), 0.0)


def _bn_fold(s, q, gamma, beta, count, HW, C, eps=1e-5):
    s_c = s.sum(axis=(0, 1)).reshape(HW, C).sum(axis=0)
    q_c = q.sum(axis=(0, 1)).reshape(HW, C).sum(axis=0)
    mean = s_c / count
    var = q_c / count - mean * mean
    scale = gamma * lax.rsqrt(var + eps)
    shift = beta - mean * scale
    return (jnp.tile(scale, HW).reshape(1, HW * C),
            jnp.tile(shift, HW).reshape(1, HW * C))


def kernel(x, w1_oi, w2_oi, w1x1_oi, w1, w2, w1x1,
           b1, b2, b1x1, g1, be1, g2, be2):
    N, C, D, H, W = x.shape
    L = H * W * C
    HW = H * W
    NB = _NB
    G = N // NB

    xb = jnp.transpose(x, (0, 2, 3, 4, 1)).reshape(N, D, L).astype(jnp.bfloat16)

    w3a = _band_mats(w1, H, W).reshape(3 * L, L).astype(jnp.bfloat16)
    w3b = _band_mats(w2, H, W).reshape(3 * L, L).astype(jnp.bfloat16)
    wres = jnp.kron(jnp.eye(HW, dtype=w1x1.dtype), w1x1).astype(jnp.bfloat16)

    def tile_c(v):
        return jnp.tile(v, HW).reshape(1, L)

    b1t, b2t, brt = tile_c(b1), tile_c(b2), tile_c(b1x1)

    row_spec = pl.BlockSpec((NB, D, L), lambda g: (g, 0, 0))
    vec_spec = pl.BlockSpec((1, L), lambda g: (0, 0))
    stat_spec = pl.BlockSpec((1, 1, L), lambda g: (g, 0, 0))
    w3_spec = pl.BlockSpec((3 * L, L), lambda g: (0, 0))
    wr_spec = pl.BlockSpec((L, L), lambda g: (0, 0))
    b_rows = jax.ShapeDtypeStruct((N, D, L), jnp.bfloat16)
    f_rows = jax.ShapeDtypeStruct((N, D, L), jnp.float32)
    f_stats = jax.ShapeDtypeStruct((G, 1, L), jnp.float32)
    par = pltpu.CompilerParams(dimension_semantics=("parallel",))

    y1, s1, q1 = pl.pallas_call(
        _conv_stats_body,
        out_shape=(b_rows, f_stats, f_stats),
        grid=(G,),
        in_specs=[row_spec, w3_spec, vec_spec],
        out_specs=(row_spec, stat_spec, stat_spec),
        compiler_params=par,
    )(xb, w3a, b1t)

    sc1, sh1 = _bn_fold(s1, q1, g1, be1, N * D * HW, HW, C)

    y2, s2, q2 = pl.pallas_call(
        _bnrelu_conv_stats_body,
        out_shape=(b_rows, f_stats, f_stats),
        grid=(G,),
        in_specs=[row_spec, w3_spec, vec_spec, vec_spec, vec_spec],
        out_specs=(row_spec, stat_spec, stat_spec),
        compiler_params=par,
    )(y1, w3b, b2t, sc1, sh1)

    sc2, sh2 = _bn_fold(s2, q2, g2, be2, N * D * HW, HW, C)

    outf = pl.pallas_call(
        _epilogue_body,
        out_shape=f_rows,
        grid=(G,),
        in_specs=[row_spec, row_spec, wr_spec, vec_spec, vec_spec, vec_spec],
        out_specs=row_spec,
        compiler_params=par,
    )(y2, xb, wres, brt, sc2, sh2)

    out = outf.reshape(N, D, H, W, C)
    return jnp.transpose(out, (0, 4, 1, 2, 3))


# speedup vs baseline: 3.5832x; 2.5656x over previous
import jax
import jax.numpy as jnp
from jax import lax
from jax.experimental import pallas as pl
from jax.experimental.pallas import tpu as pltpu

_NB = 4


def _band_mats(wk, H, W):
    Ci, Co = wk.shape[3], wk.shape[4]
    WC = W * Ci
    wi = jnp.arange(W)[:, None]
    wo = jnp.arange(W)[None, :]
    dw = wi - wo + 1
    okw = ((dw >= 0) & (dw <= 2)).astype(wk.dtype)
    q = wk[:, :, jnp.clip(dw, 0, 2)]
    q = q * okw[None, None, :, :, None, None]
    q = jnp.transpose(q, (0, 1, 2, 4, 3, 5)).reshape(3, 3, WC, W * Co)
    qz = jnp.concatenate([q, jnp.zeros((3, 1, WC, W * Co), wk.dtype)], axis=1)
    hi = jnp.arange(H)[:, None]
    ho = jnp.arange(H)[None, :]
    dh = hi - ho + 1
    idx = jnp.where((dh >= 0) & (dh <= 2), dh, 3)
    m = qz[:, idx]
    m = jnp.transpose(m, (0, 1, 3, 2, 4))
    return m.reshape(3, H * WC, H * W * Co)


def _tap_cat(a):
    NB, D, L = a.shape
    z = jnp.zeros((NB, 1, L), a.dtype)
    up = jnp.concatenate([z, a[:, :-1]], axis=1)
    dn = jnp.concatenate([a[:, 1:], z], axis=1)
    return jnp.concatenate([up, a, dn], axis=-1).reshape(NB * D, 3 * L)


def _conv_stats_body(xb_ref, w3_ref, b_ref, y_ref, s_ref, q_ref):
    a3 = _tap_cat(xb_ref[...])
    acc = jnp.dot(a3, w3_ref[...], preferred_element_type=jnp.float32)
    acc = acc + b_ref[...]
    s_ref[0] = jnp.sum(acc, axis=0, keepdims=True)
    q_ref[0] = jnp.sum(acc * acc, axis=0, keepdims=True)
    NB, D, L = y_ref.shape
    y_ref[...] = acc.reshape(NB, D, L).astype(y_ref.dtype)


def _bnrelu_conv_stats_body(y1_ref, w3_ref, b_ref, sc_ref, sh_ref,
                            y_ref, s_ref, q_ref):
    a = jnp.maximum(y1_ref[...].astype(jnp.float32) * sc_ref[...] + sh_ref[...],
                    0.0).astype(jnp.bfloat16)
    a3 = _tap_cat(a)
    acc = jnp.dot(a3, w3_ref[...], preferred_element_type=jnp.float32)
    acc = acc + b_ref[...]
    s_ref[0] = jnp.sum(acc, axis=0, keepdims=True)
    q_ref[0] = jnp.sum(acc * acc, axis=0, keepdims=True)
    NB, D, L = y_ref.shape
    y_ref[...] = acc.reshape(NB, D, L).astype(y_ref.dtype)


def _epilogue_body(y2_ref, xb_ref, wr_ref, br_ref, sc_ref, sh_ref, o_ref):
    NB, D, L = o_ref.shape
    a2 = jnp.maximum(y2_ref[...].astype(jnp.float32) * sc_ref[...] + sh_ref[...],
                     0.0)
    res = jnp.dot(xb_ref[...].reshape(NB * D, L), wr_ref[...],
                  preferred_element_type=jnp.float32) + br_ref[...]
    o_ref[...] = jnp.maximum(a2 + res.reshape(NB, D, L), 0.0)


def _bn_fold(s, q, gamma, beta, count, HW, C, eps=1e-5):
    s_c = s.sum(axis=(0, 1)).reshape(HW, C).sum(axis=0)
    q_c = q.sum(axis=(0, 1)).reshape(HW, C).sum(axis=0)
    mean = s_c / count
    var = q_c / count - mean * mean
    scale = gamma * lax.rsqrt(var + eps)
    shift = beta - mean * scale
    return (jnp.tile(scale, HW).reshape(1, HW * C),
            jnp.tile(shift, HW).reshape(1, HW * C))


def kernel(x, w1_oi, w2_oi, w1x1_oi, w1, w2, w1x1,
           b1, b2, b1x1, g1, be1, g2, be2):
    N, C, D, H, W = x.shape
    L = H * W * C
    HW = H * W
    NB = _NB
    G = N // NB

    xb = jnp.transpose(x, (0, 2, 3, 4, 1)).reshape(N, D, L).astype(jnp.bfloat16)

    w3a = _band_mats(w1, H, W).reshape(3 * L, L).astype(jnp.bfloat16)
    w3b = _band_mats(w2, H, W).reshape(3 * L, L).astype(jnp.bfloat16)
    wres = jnp.kron(jnp.eye(HW, dtype=w1x1.dtype), w1x1).astype(jnp.bfloat16)

    def tile_c(v):
        return jnp.tile(v, HW).reshape(1, L)

    b1t, b2t, brt = tile_c(b1), tile_c(b2), tile_c(b1x1)

    row_spec = pl.BlockSpec((NB, D, L), lambda g: (g, 0, 0))
    vec_spec = pl.BlockSpec((1, L), lambda g: (0, 0))
    stat_spec = pl.BlockSpec((1, 1, L), lambda g: (g, 0, 0))
    w3_spec = pl.BlockSpec((3 * L, L), lambda g: (0, 0))
    wr_spec = pl.BlockSpec((L, L), lambda g: (0, 0))
    b_rows = jax.ShapeDtypeStruct((N, D, L), jnp.bfloat16)
    f_rows = jax.ShapeDtypeStruct((N, D, L), jnp.float32)
    f_stats = jax.ShapeDtypeStruct((G, 1, L), jnp.float32)
    par = pltpu.CompilerParams(dimension_semantics=("parallel",))

    y1, s1, q1 = pl.pallas_call(
        _conv_stats_body,
        out_shape=(b_rows, f_stats, f_stats),
        grid=(G,),
        in_specs=[row_spec, w3_spec, vec_spec],
        out_specs=(row_spec, stat_spec, stat_spec),
        compiler_params=par,
    )(xb, w3a, b1t)

    sc1, sh1 = _bn_fold(s1, q1, g1, be1, N * D * HW, HW, C)

    y2, s2, q2 = pl.pallas_call(
        _bnrelu_conv_stats_body,
        out_shape=(b_rows, f_stats, f_stats),
        grid=(G,),
        in_specs=[row_spec, w3_spec, vec_spec, vec_spec, vec_spec],
        out_specs=(row_spec, stat_spec, stat_spec),
        compiler_params=par,
    )(y1, w3b, b2t, sc1, sh1)

    sc2, sh2 = _bn_fold(s2, q2, g2, be2, N * D * HW, HW, C)

    outf = pl.pallas_call(
        _epilogue_body,
        out_shape=f_rows,
        grid=(G,),
        in_specs=[row_spec, row_spec, wr_spec, vec_spec, vec_spec, vec_spec],
        out_specs=row_spec,
        compiler_params=par,
    )(y2, xb, wres, brt, sc2, sh2)

    out = outf.reshape(N, D, H, W, C)
    return jnp.transpose(out, (0, 4, 1, 2, 3))


# speedup vs baseline: 4.8194x; 1.3450x over previous
import jax
import jax.numpy as jnp
from jax import lax
from jax.experimental import pallas as pl
from jax.experimental.pallas import tpu as pltpu

_NB = 4


def _wband_tiles(wk, W, C):
    WC = W * C
    wi = jnp.arange(W)[:, None]
    wo = jnp.arange(W)[None, :]
    dw = wi - wo + 1
    okw = ((dw >= 0) & (dw <= 2)).astype(wk.dtype)
    q = wk[:, :, jnp.clip(dw, 0, 2)]
    q = q * okw[None, None, :, :, None, None]
    q = jnp.transpose(q, (0, 1, 2, 4, 3, 5)).reshape(3, 3, WC, WC)
    qz = jnp.concatenate([q, jnp.zeros((3, 1, WC, WC), wk.dtype)], axis=1)
    return qz.astype(jnp.bfloat16)


def _tap_cat(a):
    NB, D, L = a.shape
    z = jnp.zeros((NB, 1, L), a.dtype)
    up = jnp.concatenate([z, a[:, :-1]], axis=1)
    dn = jnp.concatenate([a[:, 1:], z], axis=1)
    return jnp.concatenate([up, a, dn], axis=-1).reshape(NB * D, 3 * L)


def _chan_pattern(C, L):
    l = lax.broadcasted_iota(jnp.int32, (C, L), 1)
    c = lax.broadcasted_iota(jnp.int32, (C, L), 0)
    return (l % C == c).astype(jnp.float32)


def kernel(x, w1_oi, w2_oi, w1x1_oi, w1, w2, w1x1,
           b1, b2, b1x1, g1, be1, g2, be2):
    N, C, D, H, W = x.shape
    L = H * W * C
    NB = _NB
    G = N // NB
    M = NB * D
    count = float(N * D * H * W)

    xb = jnp.transpose(x, (0, 2, 3, 4, 1)).reshape(N, D, L).astype(jnp.bfloat16)

    qa = _wband_tiles(w1, W, C)
    qb = _wband_tiles(w2, W, C)
    b128 = jnp.kron(jnp.eye(W, dtype=w1x1.dtype), w1x1).astype(jnp.bfloat16)
    vecs = jnp.stack([b1, b2, b1x1, g1, be1, g2, be2], axis=0)

    def body(xb_ref, qa_ref, qb_ref, b128_ref, vecs_ref, o_ref,
             y1_scr, y2_scr, wa_scr, wb_scr, vt_scr, st_scr, bn_scr):
        s = pl.program_id(0)
        g = pl.program_id(1)

        @pl.when(jnp.logical_and(s == 0, g == 0))
        def _prep():
            P = _chan_pattern(C, L)
            vt_scr[0:7, :] = jnp.dot(vecs_ref[...], P,
                                     preferred_element_type=jnp.float32)
            st_scr[...] = jnp.zeros_like(st_scr)
            for kd in range(3):
                for hi in range(H):
                    ta, tb = [], []
                    for ho in range(H):
                        dh = hi - ho + 1
                        d = dh if 0 <= dh <= 2 else 3
                        ta.append(qa_ref[kd, d])
                        tb.append(qb_ref[kd, d])
                    r = kd * L + hi * (W * C)
                    wa_scr[r:r + W * C, :] = jnp.concatenate(ta, axis=1)
                    wb_scr[r:r + W * C, :] = jnp.concatenate(tb, axis=1)

        @pl.when(s == 0)
        def _stage1():
            a3 = _tap_cat(xb_ref[...])
            acc = jnp.dot(a3, wa_scr[...], preferred_element_type=jnp.float32)
            acc = acc + vt_scr[0:1]
            st_scr[0:1] += jnp.sum(acc, axis=0, keepdims=True)
            st_scr[1:2] += jnp.sum(acc * acc, axis=0, keepdims=True)
            y1_scr[pl.ds(g * NB, NB)] = acc.reshape(NB, D, L).astype(jnp.bfloat16)

        def _fold(s_row, gam_row, out_row):
            P = _chan_pattern(C, L)
            sq = jnp.dot(st_scr[s_row:s_row + 2], P.T,
                         preferred_element_type=jnp.float32)
            mean = sq[0:1] / count
            var = sq[1:2] / count - mean * mean
            scale = vecs_ref[gam_row:gam_row + 1] * lax.rsqrt(var + 1e-5)
            shift = vecs_ref[gam_row + 1:gam_row + 2] - mean * scale
            bn_scr[out_row:out_row + 2] = jnp.dot(
                jnp.concatenate([scale, shift], axis=0), P,
                preferred_element_type=jnp.float32)

        @pl.when(jnp.logical_and(s == 1, g == 0))
        def _fold1():
            _fold(0, 3, 0)

        @pl.when(s == 1)
        def _stage2():
            y1 = y1_scr[pl.ds(g * NB, NB)].astype(jnp.float32)
            a = jnp.maximum(y1 * bn_scr[0:1] + bn_scr[1:2],
                            0.0).astype(jnp.bfloat16)
            a3 = _tap_cat(a)
            acc = jnp.dot(a3, wb_scr[...], preferred_element_type=jnp.float32)
            acc = acc + vt_scr[1:2]
            st_scr[2:3] += jnp.sum(acc, axis=0, keepdims=True)
            st_scr[3:4] += jnp.sum(acc * acc, axis=0, keepdims=True)
            y2_scr[pl.ds(g * NB, NB)] = acc.reshape(NB, D, L).astype(jnp.bfloat16)

        @pl.when(jnp.logical_and(s == 2, g == 0))
        def _fold2():
            _fold(2, 5, 2)

        @pl.when(s == 2)
        def _epilogue():
            y2 = y2_scr[pl.ds(g * NB, NB)].astype(jnp.float32)
            a2 = jnp.maximum(y2 * bn_scr[2:3] + bn_scr[3:4], 0.0)
            xf = xb_ref[...].reshape(M, L)
            WC = W * C
            res = jnp.concatenate(
                [jnp.dot(xf[:, j * WC:(j + 1) * WC], b128_ref[...],
                         preferred_element_type=jnp.float32)
                 for j in range(L // WC)], axis=1)
            res = res + vt_scr[2:3]
            o_ref[...] = jnp.maximum(a2 + res.reshape(NB, D, L), 0.0)

    outf = pl.pallas_call(
        body,
        out_shape=jax.ShapeDtypeStruct((N, D, L), jnp.float32),
        grid=(3, G),
        in_specs=[
            pl.BlockSpec((NB, D, L),
                         lambda s, g: (jnp.where(s == 1, 0, g), 0, 0)),
            pl.BlockSpec((3, 4, W * C, W * C), lambda s, g: (0, 0, 0, 0)),
            pl.BlockSpec((3, 4, W * C, W * C), lambda s, g: (0, 0, 0, 0)),
            pl.BlockSpec((W * C, W * C), lambda s, g: (0, 0)),
            pl.BlockSpec((7, C), lambda s, g: (0, 0)),
        ],
        out_specs=pl.BlockSpec((NB, D, L),
                               lambda s, g: (jnp.where(s == 2, g, 0), 0, 0)),
        scratch_shapes=[
            pltpu.VMEM((N, D, L), jnp.bfloat16),
            pltpu.VMEM((N, D, L), jnp.bfloat16),
            pltpu.VMEM((3 * L, L), jnp.bfloat16),
            pltpu.VMEM((3 * L, L), jnp.bfloat16),
            pltpu.VMEM((8, L), jnp.float32),
            pltpu.VMEM((4, L), jnp.float32),
            pltpu.VMEM((4, L), jnp.float32),
        ],
        compiler_params=pltpu.CompilerParams(
            dimension_semantics=("arbitrary", "arbitrary"),
            vmem_limit_bytes=60 * 1024 * 1024,
        ),
    )(xb, qa, qb, b128, vecs)

    out = outf.reshape(N, D, H, W, C)
    return jnp.transpose(out, (0, 4, 1, 2, 3))


# speedup vs baseline: 5.0381x; 1.0454x over previous
import jax
import jax.numpy as jnp
from jax import lax
from jax.experimental import pallas as pl
from jax.experimental.pallas import tpu as pltpu

_NB = 8


def _wband_tiles(wk, W, C):
    WC = W * C
    wi = jnp.arange(W)[:, None]
    wo = jnp.arange(W)[None, :]
    dw = wi - wo + 1
    okw = ((dw >= 0) & (dw <= 2)).astype(wk.dtype)
    q = wk[:, :, jnp.clip(dw, 0, 2)]
    q = q * okw[None, None, :, :, None, None]
    q = jnp.transpose(q, (0, 1, 2, 4, 3, 5)).reshape(3, 3, WC, WC)
    qz = jnp.concatenate([q, jnp.zeros((3, 1, WC, WC), wk.dtype)], axis=1)
    return qz.astype(jnp.bfloat16)


def _tap_cat(a):
    NB, D, L = a.shape
    z = jnp.zeros((NB, 1, L), a.dtype)
    up = jnp.concatenate([z, a[:, :-1]], axis=1)
    dn = jnp.concatenate([a[:, 1:], z], axis=1)
    return jnp.concatenate([up, a, dn], axis=-1).reshape(NB * D, 3 * L)


def _chan_pattern(C, L):
    l = lax.broadcasted_iota(jnp.int32, (C, L), 1)
    c = lax.broadcasted_iota(jnp.int32, (C, L), 0)
    return (l % C == c).astype(jnp.float32)


def kernel(x, w1_oi, w2_oi, w1x1_oi, w1, w2, w1x1,
           b1, b2, b1x1, g1, be1, g2, be2):
    N, C, D, H, W = x.shape
    L = H * W * C
    NB = _NB
    G = N // NB
    M = NB * D
    count = float(N * D * H * W)

    xb = jnp.transpose(x, (0, 2, 3, 4, 1)).reshape(N, D, L).astype(jnp.bfloat16)

    qa = _wband_tiles(w1, W, C)
    qb = _wband_tiles(w2, W, C)
    b128 = jnp.kron(jnp.eye(W, dtype=w1x1.dtype), w1x1).astype(jnp.bfloat16)
    vecs = jnp.stack([b1, b2, b1x1, g1, be1, g2, be2], axis=0)

    def body(xb_ref, qa_ref, qb_ref, b128_ref, vecs_ref, o_ref,
             y1_scr, y2_scr, wa_scr, wb_scr, vt_scr, st_scr, bn_scr):
        s = pl.program_id(0)
        g = pl.program_id(1)

        @pl.when(jnp.logical_and(s == 0, g == 0))
        def _prep():
            P = _chan_pattern(C, L)
            vt_scr[0:7, :] = jnp.dot(vecs_ref[...], P,
                                     preferred_element_type=jnp.float32)
            st_scr[...] = jnp.zeros_like(st_scr)
            for kd in range(3):
                for hi in range(H):
                    ta, tb = [], []
                    for ho in range(H):
                        dh = hi - ho + 1
                        d = dh if 0 <= dh <= 2 else 3
                        ta.append(qa_ref[kd, d])
                        tb.append(qb_ref[kd, d])
                    r = kd * L + hi * (W * C)
                    wa_scr[r:r + W * C, :] = jnp.concatenate(ta, axis=1)
                    wb_scr[r:r + W * C, :] = jnp.concatenate(tb, axis=1)

        @pl.when(s == 0)
        def _stage1():
            a3 = _tap_cat(xb_ref[...])
            acc = jnp.dot(a3, wa_scr[...], preferred_element_type=jnp.float32)
            acc = acc + vt_scr[0:1]
            st_scr[0:1] += jnp.sum(acc, axis=0, keepdims=True)
            st_scr[1:2] += jnp.sum(acc * acc, axis=0, keepdims=True)
            y1_scr[pl.ds(g * NB, NB)] = acc.reshape(NB, D, L).astype(jnp.bfloat16)

        def _fold(s_row, gam_row, out_row):
            P = _chan_pattern(C, L)
            sq = jnp.dot(st_scr[s_row:s_row + 2], P.T,
                         preferred_element_type=jnp.float32)
            mean = sq[0:1] / count
            var = sq[1:2] / count - mean * mean
            scale = vecs_ref[gam_row:gam_row + 1] * lax.rsqrt(var + 1e-5)
            shift = vecs_ref[gam_row + 1:gam_row + 2] - mean * scale
            bn_scr[out_row:out_row + 2] = jnp.dot(
                jnp.concatenate([scale, shift], axis=0), P,
                preferred_element_type=jnp.float32)

        @pl.when(jnp.logical_and(s == 1, g == 0))
        def _fold1():
            _fold(0, 3, 0)

        @pl.when(s == 1)
        def _stage2():
            y1 = y1_scr[pl.ds(g * NB, NB)].astype(jnp.float32)
            a = jnp.maximum(y1 * bn_scr[0:1] + bn_scr[1:2],
                            0.0).astype(jnp.bfloat16)
            a3 = _tap_cat(a)
            acc = jnp.dot(a3, wb_scr[...], preferred_element_type=jnp.float32)
            acc = acc + vt_scr[1:2]
            st_scr[2:3] += jnp.sum(acc, axis=0, keepdims=True)
            st_scr[3:4] += jnp.sum(acc * acc, axis=0, keepdims=True)
            y2_scr[pl.ds(g * NB, NB)] = acc.reshape(NB, D, L).astype(jnp.bfloat16)

        @pl.when(jnp.logical_and(s == 2, g == 0))
        def _fold2():
            _fold(2, 5, 2)

        @pl.when(s == 2)
        def _epilogue():
            y2 = y2_scr[pl.ds(g * NB, NB)].astype(jnp.float32)
            a2 = jnp.maximum(y2 * bn_scr[2:3] + bn_scr[3:4], 0.0)
            xf = xb_ref[...].reshape(M, L)
            WC = W * C
            res = jnp.concatenate(
                [jnp.dot(xf[:, j * WC:(j + 1) * WC], b128_ref[...],
                         preferred_element_type=jnp.float32)
                 for j in range(L // WC)], axis=1)
            res = res + vt_scr[2:3]
            o_ref[...] = jnp.maximum(a2 + res.reshape(NB, D, L), 0.0)

    outf = pl.pallas_call(
        body,
        out_shape=jax.ShapeDtypeStruct((N, D, L), jnp.float32),
        grid=(3, G),
        in_specs=[
            pl.BlockSpec((NB, D, L),
                         lambda s, g: (jnp.where(s == 1, 0, g), 0, 0)),
            pl.BlockSpec((3, 4, W * C, W * C), lambda s, g: (0, 0, 0, 0)),
            pl.BlockSpec((3, 4, W * C, W * C), lambda s, g: (0, 0, 0, 0)),
            pl.BlockSpec((W * C, W * C), lambda s, g: (0, 0)),
            pl.BlockSpec((7, C), lambda s, g: (0, 0)),
        ],
        out_specs=pl.BlockSpec((NB, D, L),
                               lambda s, g: (jnp.where(s == 2, g, 0), 0, 0)),
        scratch_shapes=[
            pltpu.VMEM((N, D, L), jnp.bfloat16),
            pltpu.VMEM((N, D, L), jnp.bfloat16),
            pltpu.VMEM((3 * L, L), jnp.bfloat16),
            pltpu.VMEM((3 * L, L), jnp.bfloat16),
            pltpu.VMEM((8, L), jnp.float32),
            pltpu.VMEM((4, L), jnp.float32),
            pltpu.VMEM((4, L), jnp.float32),
        ],
        compiler_params=pltpu.CompilerParams(
            dimension_semantics=("arbitrary", "arbitrary"),
            vmem_limit_bytes=60 * 1024 * 1024,
        ),
    )(xb, qa, qb, b128, vecs)

    out = outf.reshape(N, D, H, W, C)
    return jnp.transpose(out, (0, 4, 1, 2, 3))


# speedup vs baseline: 5.1482x; 1.0218x over previous
import jax
import jax.numpy as jnp
from jax import lax
from jax.experimental import pallas as pl
from jax.experimental.pallas import tpu as pltpu

_NB = 16


def _wband_tiles(wk, W, C):
    WC = W * C
    wi = jnp.arange(W)[:, None]
    wo = jnp.arange(W)[None, :]
    dw = wi - wo + 1
    okw = ((dw >= 0) & (dw <= 2)).astype(wk.dtype)
    q = wk[:, :, jnp.clip(dw, 0, 2)]
    q = q * okw[None, None, :, :, None, None]
    q = jnp.transpose(q, (0, 1, 2, 4, 3, 5)).reshape(3, 3, WC, WC)
    qz = jnp.concatenate([q, jnp.zeros((3, 1, WC, WC), wk.dtype)], axis=1)
    return qz.astype(jnp.bfloat16)


def _tap_cat(a):
    NB, D, L = a.shape
    z = jnp.zeros((NB, 1, L), a.dtype)
    up = jnp.concatenate([z, a[:, :-1]], axis=1)
    dn = jnp.concatenate([a[:, 1:], z], axis=1)
    return jnp.concatenate([up, a, dn], axis=-1).reshape(NB * D, 3 * L)


def _chan_pattern(C, L):
    l = lax.broadcasted_iota(jnp.int32, (C, L), 1)
    c = lax.broadcasted_iota(jnp.int32, (C, L), 0)
    return (l % C == c).astype(jnp.float32)


def kernel(x, w1_oi, w2_oi, w1x1_oi, w1, w2, w1x1,
           b1, b2, b1x1, g1, be1, g2, be2):
    N, C, D, H, W = x.shape
    L = H * W * C
    NB = _NB
    G = N // NB
    M = NB * D
    count = float(N * D * H * W)

    xb = jnp.transpose(x, (0, 2, 3, 4, 1)).reshape(N, D, L).astype(jnp.bfloat16)

    qa = _wband_tiles(w1, W, C)
    qb = _wband_tiles(w2, W, C)
    b128 = jnp.kron(jnp.eye(W, dtype=w1x1.dtype), w1x1).astype(jnp.bfloat16)
    vecs = jnp.stack([b1, b2, b1x1, g1, be1, g2, be2], axis=0)

    def body(xb_ref, qa_ref, qb_ref, b128_ref, vecs_ref, o_ref,
             y1_scr, y2_scr, wa_scr, wb_scr, vt_scr, st_scr, bn_scr):
        s = pl.program_id(0)
        g = pl.program_id(1)

        @pl.when(jnp.logical_and(s == 0, g == 0))
        def _prep():
            P = _chan_pattern(C, L)
            vt_scr[0:7, :] = jnp.dot(vecs_ref[...], P,
                                     preferred_element_type=jnp.float32)
            st_scr[...] = jnp.zeros_like(st_scr)
            for kd in range(3):
                for hi in range(H):
                    ta, tb = [], []
                    for ho in range(H):
                        dh = hi - ho + 1
                        d = dh if 0 <= dh <= 2 else 3
                        ta.append(qa_ref[kd, d])
                        tb.append(qb_ref[kd, d])
                    r = kd * L + hi * (W * C)
                    wa_scr[r:r + W * C, :] = jnp.concatenate(ta, axis=1)
                    wb_scr[r:r + W * C, :] = jnp.concatenate(tb, axis=1)

        @pl.when(s == 0)
        def _stage1():
            a3 = _tap_cat(xb_ref[...])
            acc = jnp.dot(a3, wa_scr[...], preferred_element_type=jnp.float32)
            acc = acc + vt_scr[0:1]
            st_scr[0:1] += jnp.sum(acc, axis=0, keepdims=True)
            st_scr[1:2] += jnp.sum(acc * acc, axis=0, keepdims=True)
            y1_scr[pl.ds(g * NB, NB)] = acc.reshape(NB, D, L).astype(jnp.bfloat16)

        def _fold(s_row, gam_row, out_row):
            P = _chan_pattern(C, L)
            sq = jnp.dot(st_scr[s_row:s_row + 2], P.T,
                         preferred_element_type=jnp.float32)
            mean = sq[0:1] / count
            var = sq[1:2] / count - mean * mean
            scale = vecs_ref[gam_row:gam_row + 1] * lax.rsqrt(var + 1e-5)
            shift = vecs_ref[gam_row + 1:gam_row + 2] - mean * scale
            bn_scr[out_row:out_row + 2] = jnp.dot(
                jnp.concatenate([scale, shift], axis=0), P,
                preferred_element_type=jnp.float32)

        @pl.when(jnp.logical_and(s == 1, g == 0))
        def _fold1():
            _fold(0, 3, 0)

        @pl.when(s == 1)
        def _stage2():
            y1 = y1_scr[pl.ds(g * NB, NB)].astype(jnp.float32)
            a = jnp.maximum(y1 * bn_scr[0:1] + bn_scr[1:2],
                            0.0).astype(jnp.bfloat16)
            a3 = _tap_cat(a)
            acc = jnp.dot(a3, wb_scr[...], preferred_element_type=jnp.float32)
            acc = acc + vt_scr[1:2]
            st_scr[2:3] += jnp.sum(acc, axis=0, keepdims=True)
            st_scr[3:4] += jnp.sum(acc * acc, axis=0, keepdims=True)
            y2_scr[pl.ds(g * NB, NB)] = acc.reshape(NB, D, L).astype(jnp.bfloat16)

        @pl.when(jnp.logical_and(s == 2, g == 0))
        def _fold2():
            _fold(2, 5, 2)

        @pl.when(s == 2)
        def _epilogue():
            y2 = y2_scr[pl.ds(g * NB, NB)].astype(jnp.float32)
            a2 = jnp.maximum(y2 * bn_scr[2:3] + bn_scr[3:4], 0.0)
            xf = xb_ref[...].reshape(M, L)
            WC = W * C
            res = jnp.concatenate(
                [jnp.dot(xf[:, j * WC:(j + 1) * WC], b128_ref[...],
                         preferred_element_type=jnp.float32)
                 for j in range(L // WC)], axis=1)
            res = res + vt_scr[2:3]
            o_ref[...] = jnp.maximum(a2 + res.reshape(NB, D, L), 0.0)

    outf = pl.pallas_call(
        body,
        out_shape=jax.ShapeDtypeStruct((N, D, L), jnp.float32),
        grid=(3, G),
        in_specs=[
            pl.BlockSpec((NB, D, L),
                         lambda s, g: (jnp.where(s == 1, 0, g), 0, 0)),
            pl.BlockSpec((3, 4, W * C, W * C), lambda s, g: (0, 0, 0, 0)),
            pl.BlockSpec((3, 4, W * C, W * C), lambda s, g: (0, 0, 0, 0)),
            pl.BlockSpec((W * C, W * C), lambda s, g: (0, 0)),
            pl.BlockSpec((7, C), lambda s, g: (0, 0)),
        ],
        out_specs=pl.BlockSpec((NB, D, L),
                               lambda s, g: (jnp.where(s == 2, g, 0), 0, 0)),
        scratch_shapes=[
            pltpu.VMEM((N, D, L), jnp.bfloat16),
            pltpu.VMEM((N, D, L), jnp.bfloat16),
            pltpu.VMEM((3 * L, L), jnp.bfloat16),
            pltpu.VMEM((3 * L, L), jnp.bfloat16),
            pltpu.VMEM((8, L), jnp.float32),
            pltpu.VMEM((4, L), jnp.float32),
            pltpu.VMEM((4, L), jnp.float32),
        ],
        compiler_params=pltpu.CompilerParams(
            dimension_semantics=("arbitrary", "arbitrary"),
            vmem_limit_bytes=60 * 1024 * 1024,
        ),
    )(xb, qa, qb, b128, vecs)

    out = outf.reshape(N, D, H, W, C)
    return jnp.transpose(out, (0, 4, 1, 2, 3))


# speedup vs baseline: 5.9903x; 1.1636x over previous
import jax
import jax.numpy as jnp
from jax import lax
from jax.experimental import pallas as pl
from jax.experimental.pallas import tpu as pltpu

_NB = 16


def _wband_tiles(wk, W, C):
    WC = W * C
    wi = jnp.arange(W)[:, None]
    wo = jnp.arange(W)[None, :]
    dw = wi - wo + 1
    okw = ((dw >= 0) & (dw <= 2)).astype(wk.dtype)
    q = wk[:, :, jnp.clip(dw, 0, 2)]
    q = q * okw[None, None, :, :, None, None]
    q = jnp.transpose(q, (0, 1, 2, 4, 3, 5)).reshape(3, 3, WC, WC)
    qz = jnp.concatenate([q, jnp.zeros((3, 1, WC, WC), wk.dtype)], axis=1)
    return qz.astype(jnp.bfloat16)


def _shifted(a):
    NB, D, L = a.shape
    z = jnp.zeros((NB, 1, L), a.dtype)
    up = jnp.concatenate([z, a[:, :-1]], axis=1)
    dn = jnp.concatenate([a[:, 1:], z], axis=1)
    return up, dn


def _chan_pattern(C, L):
    l = lax.broadcasted_iota(jnp.int32, (C, L), 1)
    c = lax.broadcasted_iota(jnp.int32, (C, L), 0)
    return (l % C == c).astype(jnp.float32)


def kernel(x, w1_oi, w2_oi, w1x1_oi, w1, w2, w1x1,
           b1, b2, b1x1, g1, be1, g2, be2):
    N, C, D, H, W = x.shape
    L = H * W * C
    WC = W * C
    NG = H // 2
    GW = 2 * WC
    KH = 4
    KG = 3 * KH * WC
    NB = min(_NB, N)
    G = N // NB
    M = NB * D
    count = float(N * D * H * W)

    xb = jnp.transpose(x, (0, 2, 3, 4, 1)).reshape(N, D, L).astype(jnp.bfloat16)

    qa = _wband_tiles(w1, W, C)
    qb = _wband_tiles(w2, W, C)
    b256 = jnp.kron(jnp.eye(2 * W, dtype=w1x1.dtype), w1x1).astype(jnp.bfloat16)
    vecs = jnp.stack([b1, b2, b1x1, g1, be1, g2, be2], axis=0)

    g_start = [min(max((2 * k - 1) * WC, 0), L - KH * WC) for k in range(NG)]

    def body(xb_ref, qa_ref, qb_ref, b256_ref, vecs_ref, o_ref,
             y1_scr, y2_scr, wa_scr, wb_scr, vt_scr, st_scr, bn_scr):
        s = pl.program_id(0)
        g = pl.program_id(1)

        @pl.when(jnp.logical_and(s == 0, g == 0))
        def _prep():
            P = _chan_pattern(C, L)
            vt_scr[0:7, :] = jnp.dot(vecs_ref[...], P,
                                     preferred_element_type=jnp.float32)
            st_scr[...] = jnp.zeros_like(st_scr)
            zt = jnp.zeros((WC, WC), jnp.bfloat16)
            for k in range(NG):
                hi0 = g_start[k] // WC
                for kd in range(3):
                    for hr in range(KH):
                        hi = hi0 + hr
                        ta, tb = [], []
                        for ho in (2 * k, 2 * k + 1):
                            dh = hi - ho + 1
                            if 0 <= dh <= 2:
                                ta.append(qa_ref[kd, dh])
                                tb.append(qb_ref[kd, dh])
                            else:
                                ta.append(zt)
                                tb.append(zt)
                        r = (kd * KH + hr) * WC
                        wa_scr[k, r:r + WC, :] = jnp.concatenate(ta, axis=1)
                        wb_scr[k, r:r + WC, :] = jnp.concatenate(tb, axis=1)

        def _conv_groups(a, w_scr):
            up, dn = _shifted(a)
            a2d = a.reshape(M, L)
            up2d = up.reshape(M, L)
            dn2d = dn.reshape(M, L)
            outs = []
            for k in range(NG):
                st = g_start[k]
                lhs = jnp.concatenate(
                    [up2d[:, st:st + KH * WC],
                     a2d[:, st:st + KH * WC],
                     dn2d[:, st:st + KH * WC]], axis=1)
                outs.append(jnp.dot(lhs, w_scr[k],
                                    preferred_element_type=jnp.float32))
            return outs

        def _stage_store(accs, b_row, s_row, y_scr):
            for k in range(NG):
                acc = accs[k] + vt_scr[b_row:b_row + 1, k * GW:(k + 1) * GW]
                st_scr[s_row:s_row + 1, k * GW:(k + 1) * GW] += \
                    jnp.sum(acc, axis=0, keepdims=True)
                st_scr[s_row + 1:s_row + 2, k * GW:(k + 1) * GW] += \
                    jnp.sum(acc * acc, axis=0, keepdims=True)
                y_scr[pl.ds(g * NB, NB), :, k * GW:(k + 1) * GW] = \
                    acc.reshape(NB, D, GW).astype(jnp.bfloat16)

        @pl.when(s == 0)
        def _stage1():
            _stage_store(_conv_groups(xb_ref[...], wa_scr), 0, 0, y1_scr)

        def _fold(s_row, gam_row, out_row):
            P = _chan_pattern(C, L)
            sq = jnp.dot(st_scr[s_row:s_row + 2], P.T,
                         preferred_element_type=jnp.float32)
            mean = sq[0:1] / count
            var = sq[1:2] / count - mean * mean
            scale = vecs_ref[gam_row:gam_row + 1] * lax.rsqrt(var + 1e-5)
            shift = vecs_ref[gam_row + 1:gam_row + 2] - mean * scale
            bn_scr[out_row:out_row + 2] = jnp.dot(
                jnp.concatenate([scale, shift], axis=0), P,
                preferred_element_type=jnp.float32)

        @pl.when(jnp.logical_and(s == 1, g == 0))
        def _fold1():
            _fold(0, 3, 0)

        @pl.when(s == 1)
        def _stage2():
            y1 = y1_scr[pl.ds(g * NB, NB)].astype(jnp.float32)
            a = jnp.maximum(y1 * bn_scr[0:1] + bn_scr[1:2],
                            0.0).astype(jnp.bfloat16)
            _stage_store(_conv_groups(a, wb_scr), 1, 2, y2_scr)

        @pl.when(jnp.logical_and(s == 2, g == 0))
        def _fold2():
            _fold(2, 5, 2)

        @pl.when(s == 2)
        def _epilogue():
            y2 = y2_scr[pl.ds(g * NB, NB)].astype(jnp.float32)
            a2 = jnp.maximum(y2 * bn_scr[2:3] + bn_scr[3:4], 0.0)
            xf = xb_ref[...].reshape(M, L)
            res = jnp.concatenate(
                [jnp.dot(xf[:, k * GW:(k + 1) * GW], b256_ref[...],
                         preferred_element_type=jnp.float32)
                 for k in range(L // GW)], axis=1)
            res = res + vt_scr[2:3]
            o_ref[...] = jnp.maximum(a2 + res.reshape(NB, D, L), 0.0)

    outf = pl.pallas_call(
        body,
        out_shape=jax.ShapeDtypeStruct((N, D, L), jnp.float32),
        grid=(3, G),
        in_specs=[
            pl.BlockSpec((NB, D, L),
                         lambda s, g: (jnp.where(s == 1, 0, g), 0, 0)),
            pl.BlockSpec((3, 4, WC, WC), lambda s, g: (0, 0, 0, 0)),
            pl.BlockSpec((3, 4, WC, WC), lambda s, g: (0, 0, 0, 0)),
            pl.BlockSpec((GW, GW), lambda s, g: (0, 0)),
            pl.BlockSpec((7, C), lambda s, g: (0, 0)),
        ],
        out_specs=pl.BlockSpec((NB, D, L),
                               lambda s, g: (jnp.where(s == 2, g, 0), 0, 0)),
        scratch_shapes=[
            pltpu.VMEM((N, D, L), jnp.bfloat16),
            pltpu.VMEM((N, D, L), jnp.bfloat16),
            pltpu.VMEM((NG, KG, GW), jnp.bfloat16),
            pltpu.VMEM((NG, KG, GW), jnp.bfloat16),
            pltpu.VMEM((8, L), jnp.float32),
            pltpu.VMEM((4, L), jnp.float32),
            pltpu.VMEM((4, L), jnp.float32),
        ],
        compiler_params=pltpu.CompilerParams(
            dimension_semantics=("arbitrary", "arbitrary"),
            vmem_limit_bytes=60 * 1024 * 1024,
        ),
    )(xb, qa, qb, b256, vecs)

    out = outf.reshape(N, D, H, W, C)
    return jnp.transpose(out, (0, 4, 1, 2, 3))
